# baseline (device time: 18180 ns/iter reference)
import jax
import jax.numpy as jnp
from jax import lax
from jax.experimental import pallas as pl
from jax.experimental.pallas import tpu as pltpu

N_DEV = 4
N_RINGS = 4


def _gelu(y):
    c = 0.7978845608028654
    return 0.5 * y * (1.0 + jnp.tanh(c * (y + 0.044715 * y * y * y)))


def kernel(x, w_mat):
    m, k = x.shape
    _, n = w_mat.shape
    m_out = m // N_DEV
    nq = n // N_RINGS
    nh = n // 2

    def body(x_ref, w_ref, out_ref, comm_ref, send_sems, recv_sems):
        d = lax.axis_index("i")
        left = lax.rem(d + N_DEV - 1, N_DEV)
        right = lax.rem(d + 1, N_DEV)

        def mm(c, lo, width):
            return jnp.dot(
                x_ref[pl.ds(c * m_out, m_out), :],
                w_ref[:, lo:lo + width],
                preferred_element_type=jnp.float32,
            )

        barrier_sem = pltpu.get_barrier_semaphore()
        for nbr in (left, right):
            pl.semaphore_signal(
                barrier_sem, inc=1,
                device_id=(nbr,), device_id_type=pl.DeviceIdType.MESH,
            )
        pl.semaphore_wait(barrier_sem, 2)

        def start(r, s):
            rdma = pltpu.make_async_remote_copy(
                src_ref=comm_ref.at[r, s],
                dst_ref=comm_ref.at[r, s + 1],
                send_sem=send_sems.at[r, s],
                recv_sem=recv_sems.at[r, s],
                device_id=(right if r < 2 else left,),
                device_id_type=pl.DeviceIdType.MESH,
            )
            rdma.start()
            return rdma

        rdmas = []
        init_l = mm(lax.rem(d + 3, N_DEV), 0, nh)
        comm_ref[0, 0, :, :] = init_l[:, :nq].astype(jnp.bfloat16)
        rdmas.append(start(0, 0))
        comm_ref[1, 0, :, :] = init_l[:, nq:].astype(jnp.bfloat16)
        rdmas.append(start(1, 0))
        init_r = mm(lax.rem(d + 1, N_DEV), nh, nh)
        comm_ref[2, 0, :, :] = init_r[:, :nq].astype(jnp.bfloat16)
        rdmas.append(start(2, 0))
        comm_ref[3, 0, :, :] = init_r[:, nq:].astype(jnp.bfloat16)
        rdmas.append(start(3, 0))

        pre_l = mm(lax.rem(d + 2, N_DEV), 0, nh)
        pre_r = mm(lax.rem(d + 2, N_DEV), nh, nh)

        recvs = []
        for s in range(N_DEV - 1):
            for r in (0, 2, 1, 3):
                recv = pltpu.make_async_remote_copy(
                    src_ref=comm_ref.at[r, s],
                    dst_ref=comm_ref.at[r, s + 1],
                    send_sem=send_sems.at[r, s],
                    recv_sem=recv_sems.at[r, s],
                    device_id=(right if r < 2 else left,),
                    device_id_type=pl.DeviceIdType.MESH,
                )
                recv.wait_recv()
                pre = pre_l if r < 2 else pre_r
                qlo = (r % 2) * nq
                q = comm_ref[r, s + 1, :, :].astype(jnp.float32) + pre[
                    :, qlo:qlo + nq
                ]
                if s < N_DEV - 2:
                    comm_ref[r, s + 1, :, :] = q.astype(jnp.bfloat16)
                    rdmas.append(start(r, s + 1))
                else:
                    out_ref[:, pl.ds((2 * (r // 2) + (r % 2)) * nq, nq)] = (
                        _gelu(q)
                    )
            if s == 0:
                pre_l = mm(lax.rem(d + 1, N_DEV), 0, nh)
                pre_r = mm(lax.rem(d + 3, N_DEV), nh, nh)
            elif s == 1:
                pre_l = mm(d, 0, nh)
                pre_r = mm(d, nh, nh)

        for rdma in rdmas:
            rdma.wait_send()

    return pl.pallas_call(
        body,
        out_shape=jax.ShapeDtypeStruct((m_out, n), jnp.float32),
        in_specs=[
            pl.BlockSpec(memory_space=pltpu.VMEM),
            pl.BlockSpec(memory_space=pltpu.VMEM),
        ],
        out_specs=pl.BlockSpec(memory_space=pltpu.VMEM),
        scratch_shapes=[
            pltpu.VMEM((N_RINGS, N_DEV, m_out, nq), jnp.bfloat16),
            pltpu.SemaphoreType.DMA((N_RINGS, N_DEV - 1)),
            pltpu.SemaphoreType.DMA((N_RINGS, N_DEV - 1)),
        ],
        compiler_params=pltpu.CompilerParams(collective_id=0),
    )(x, w_mat)


# device time: 17637 ns/iter; 1.0308x vs baseline; 1.0308x over previous
import jax
import jax.numpy as jnp
from jax import lax
from jax.experimental import pallas as pl
from jax.experimental.pallas import tpu as pltpu

N_DEV = 4
N_MSG = 6


def _gelu(y):
    c = 0.7978845608028654
    return 0.5 * y * (1.0 + jnp.tanh(c * (y + 0.044715 * y * y * y)))


def kernel(x, w_mat):
    m, k = x.shape
    _, n = w_mat.shape
    m_out = m // N_DEV
    nh = n // 2

    TO_LEFT = (0, 2, 4)

    def body(x_ref, w_ref, out_ref, send_buf, recv_buf, send_sems, recv_sems):
        d = lax.axis_index("i")
        left = lax.rem(d + N_DEV - 1, N_DEV)
        right = lax.rem(d + 1, N_DEV)

        def mm(c, lo, width):
            return jnp.dot(
                x_ref[pl.ds(lax.rem(c, N_DEV) * m_out, m_out), :],
                w_ref[:, lo:lo + width],
                preferred_element_type=jnp.float32,
            )

        barrier_sem = pltpu.get_barrier_semaphore()
        for nbr in (left, right):
            pl.semaphore_signal(
                barrier_sem, inc=1,
                device_id=(nbr,), device_id_type=pl.DeviceIdType.MESH,
            )
        pl.semaphore_wait(barrier_sem, 2)

        def make(j):
            return pltpu.make_async_remote_copy(
                src_ref=send_buf.at[j],
                dst_ref=recv_buf.at[j],
                send_sem=send_sems.at[j],
                recv_sem=recv_sems.at[j],
                device_id=(left if j in TO_LEFT else right,),
                device_id_type=pl.DeviceIdType.MESH,
            )

        diag = mm(d + 2, 0, n)
        send_buf[0, :, :] = diag[:, :nh].astype(jnp.bfloat16)
        s0 = make(0)
        s0.start()
        send_buf[1, :, :] = diag[:, nh:].astype(jnp.bfloat16)
        s1 = make(1)
        s1.start()

        chunk_lm1 = mm(d + 3, 0, n)
        send_buf[2, :, :] = chunk_lm1[:, nh:].astype(jnp.bfloat16)
        s2 = make(2)
        s2.start()
        chunk_lp1 = mm(d + 1, 0, n)
        send_buf[3, :, :] = chunk_lp1[:, :nh].astype(jnp.bfloat16)
        s3 = make(3)
        s3.start()

        own = mm(d, 0, n)

        r0 = make(0)
        r0.wait_recv()
        send_buf[4, :, :] = (
            chunk_lm1[:, :nh] + recv_buf[0, :, :].astype(jnp.float32)
        ).astype(jnp.bfloat16)
        s4 = make(4)
        s4.start()
        r1 = make(1)
        r1.wait_recv()
        send_buf[5, :, :] = (
            chunk_lp1[:, nh:] + recv_buf[1, :, :].astype(jnp.float32)
        ).astype(jnp.bfloat16)
        s5 = make(5)
        s5.start()

        for j in (2, 3, 4, 5):
            make(j).wait_recv()
        half_a = own[:, :nh] + (
            recv_buf[4, :, :].astype(jnp.float32)
            + recv_buf[3, :, :].astype(jnp.float32)
        )
        half_b = own[:, nh:] + (
            recv_buf[5, :, :].astype(jnp.float32)
            + recv_buf[2, :, :].astype(jnp.float32)
        )
        out_ref[:, :nh] = _gelu(half_a)
        out_ref[:, nh:] = _gelu(half_b)

        for s in (s0, s1, s2, s3, s4, s5):
            s.wait_send()

    return pl.pallas_call(
        body,
        out_shape=jax.ShapeDtypeStruct((m_out, n), jnp.float32),
        in_specs=[
            pl.BlockSpec(memory_space=pltpu.VMEM),
            pl.BlockSpec(memory_space=pltpu.VMEM),
        ],
        out_specs=pl.BlockSpec(memory_space=pltpu.VMEM),
        scratch_shapes=[
            pltpu.VMEM((N_MSG, m_out, nh), jnp.bfloat16),
            pltpu.VMEM((N_MSG, m_out, nh), jnp.bfloat16),
            pltpu.SemaphoreType.DMA((N_MSG,)),
            pltpu.SemaphoreType.DMA((N_MSG,)),
        ],
        compiler_params=pltpu.CompilerParams(collective_id=0),
    )(x, w_mat)


# device time: 17259 ns/iter; 1.0534x vs baseline; 1.0219x over previous
import jax
import jax.numpy as jnp
from jax import lax
from jax.experimental import pallas as pl
from jax.experimental.pallas import tpu as pltpu

N_DEV = 4
N_MSG = 6


def _gelu(y):
    c = 0.7978845608028654
    return 0.5 * y * (1.0 + jnp.tanh(c * (y + 0.044715 * y * y * y)))


def kernel(x, w_mat):
    m, k = x.shape
    _, n = w_mat.shape
    m_out = m // N_DEV
    nh = n // 2

    TO_LEFT = (0, 2, 4)

    def body(x_ref, w_ref, out_ref, send_buf, recv_buf, send_sems, recv_sems):
        d = lax.axis_index("i")
        left = lax.rem(d + N_DEV - 1, N_DEV)
        right = lax.rem(d + 1, N_DEV)

        def mm(c, lo, width):
            return jnp.dot(
                x_ref[pl.ds(lax.rem(c, N_DEV) * m_out, m_out), :],
                w_ref[:, lo:lo + width],
                preferred_element_type=jnp.float32,
            )

        barrier_sem = pltpu.get_barrier_semaphore()
        for nbr in (left, right):
            pl.semaphore_signal(
                barrier_sem, inc=1,
                device_id=(nbr,), device_id_type=pl.DeviceIdType.MESH,
            )
        pl.semaphore_wait(barrier_sem, 2)

        def make(j):
            return pltpu.make_async_remote_copy(
                src_ref=send_buf.at[j],
                dst_ref=recv_buf.at[j],
                send_sem=send_sems.at[j],
                recv_sem=recv_sems.at[j],
                device_id=(left if j in TO_LEFT else right,),
                device_id_type=pl.DeviceIdType.MESH,
            )

        send_buf[0, :, :] = mm(d + 2, 0, nh).astype(jnp.bfloat16)
        s0 = make(0)
        s0.start()
        send_buf[1, :, :] = mm(d + 2, nh, nh).astype(jnp.bfloat16)
        s1 = make(1)
        s1.start()

        chunk_lm1 = mm(d + 3, 0, n)
        send_buf[2, :, :] = chunk_lm1[:, nh:].astype(jnp.bfloat16)
        s2 = make(2)
        s2.start()
        chunk_lp1 = mm(d + 1, 0, n)
        send_buf[3, :, :] = chunk_lp1[:, :nh].astype(jnp.bfloat16)
        s3 = make(3)
        s3.start()

        r0 = make(0)
        r0.wait_recv()
        send_buf[4, :, :] = (
            chunk_lm1[:, :nh] + recv_buf[0, :, :].astype(jnp.float32)
        ).astype(jnp.bfloat16)
        s4 = make(4)
        s4.start()
        r1 = make(1)
        r1.wait_recv()
        send_buf[5, :, :] = (
            chunk_lp1[:, nh:] + recv_buf[1, :, :].astype(jnp.float32)
        ).astype(jnp.bfloat16)
        s5 = make(5)
        s5.start()

        own = mm(d, 0, n)
        make(3).wait_recv()
        pre_a = own[:, :nh] + recv_buf[3, :, :].astype(jnp.float32)
        make(2).wait_recv()
        pre_b = own[:, nh:] + recv_buf[2, :, :].astype(jnp.float32)

        make(4).wait_recv()
        out_ref[:, :nh] = _gelu(pre_a + recv_buf[4, :, :].astype(jnp.float32))
        make(5).wait_recv()
        out_ref[:, nh:] = _gelu(pre_b + recv_buf[5, :, :].astype(jnp.float32))

        for s in (s0, s1, s2, s3, s4, s5):
            s.wait_send()

    return pl.pallas_call(
        body,
        out_shape=jax.ShapeDtypeStruct((m_out, n), jnp.float32),
        in_specs=[
            pl.BlockSpec(memory_space=pltpu.VMEM),
            pl.BlockSpec(memory_space=pltpu.VMEM),
        ],
        out_specs=pl.BlockSpec(memory_space=pltpu.VMEM),
        scratch_shapes=[
            pltpu.VMEM((N_MSG, m_out, nh), jnp.bfloat16),
            pltpu.VMEM((N_MSG, m_out, nh), jnp.bfloat16),
            pltpu.SemaphoreType.DMA((N_MSG,)),
            pltpu.SemaphoreType.DMA((N_MSG,)),
        ],
        compiler_params=pltpu.CompilerParams(collective_id=0),
    )(x, w_mat)
